# baseline (speedup 1.0000x reference)
import functools

import jax
import jax.numpy as jnp
from jax import lax
from jax.experimental import pallas as pl
from jax.experimental.pallas import tpu as pltpu
from jax.experimental.pallas import tpu_sc as plsc

VOCAB = 100000
D_MODEL = 64
BATCH = 1024

NUM_CORES = 2
NUM_SUBCORES = 16
NUM_WORKERS = NUM_CORES * NUM_SUBCORES
B_PER_W = BATCH // NUM_WORKERS

@functools.cache
def _make_gather_sc():
    mesh = plsc.VectorSubcoreMesh(core_axis_name="c", subcore_axis_name="s")

    @functools.partial(
        pl.kernel,
        mesh=mesh,
        compiler_params=pltpu.CompilerParams(use_tc_tiling_on_sc=False),
        out_type=jax.ShapeDtypeStruct((BATCH, D_MODEL), jnp.float32),
        scratch_types=[
            pltpu.VMEM((B_PER_W,), jnp.int32),
            pltpu.VMEM((B_PER_W, D_MODEL), jnp.float32),
            pltpu.SemaphoreType.DMA,
        ],
    )
    def gather_rows_sc(table_hbm, idx_hbm, out_hbm, idx_v, rows_v, sem):
        wid = lax.axis_index("s") * NUM_CORES + lax.axis_index("c")
        base = wid * B_PER_W
        pltpu.sync_copy(idx_hbm.at[pl.ds(base, B_PER_W)], idx_v)
        pltpu.async_copy(table_hbm.at[idx_v], rows_v, sem).wait()
        pltpu.sync_copy(rows_v, out_hbm.at[pl.ds(base, B_PER_W)])

    return gather_rows_sc


V_BLK = 2048
N_BLK = (VOCAB + V_BLK - 1) // V_BLK


def _proj_body(x_ref, w_ref, b_ref, out_ref):
    acc = lax.dot_general(
        x_ref[...], w_ref[...],
        (((1,), (1,)), ((), ())),
        preferred_element_type=jnp.float32,
    )
    out_ref[...] = acc + b_ref[...]


def kernel(input_ids, embed_table, W, b):
    ids = input_ids.astype(jnp.int32)
    x = _make_gather_sc()(embed_table, ids)
    x16 = x.astype(jnp.bfloat16)
    w16 = W.astype(jnp.bfloat16)
    b2 = b.reshape(1, VOCAB)
    out = pl.pallas_call(
        _proj_body,
        grid=(N_BLK,),
        in_specs=[
            pl.BlockSpec((BATCH, D_MODEL), lambda j: (0, 0)),
            pl.BlockSpec((V_BLK, D_MODEL), lambda j: (j, 0)),
            pl.BlockSpec((1, V_BLK), lambda j: (0, j)),
        ],
        out_specs=pl.BlockSpec((BATCH, V_BLK), lambda j: (0, j)),
        out_shape=jax.ShapeDtypeStruct((BATCH, VOCAB), jnp.float32),
    )(x16, w16, b2)
    return out
